# baseline (speedup 1.0000x reference)
import functools

import jax
import jax.numpy as jnp
from jax import lax
from jax.experimental import pallas as pl
from jax.experimental.pallas import tpu as pltpu
from jax.experimental.pallas import tpu_sc as plsc

NC = 2
NS = 16
NW = NC * NS
LANES = 16
GB = 80
NBUF = 4
WIN = 16
WE = WIN * GB


def _mm_body(x_ref, w_ref, o_ref):
    o_ref[...] = jnp.dot(x_ref[...], w_ref[...],
                         preferred_element_type=jnp.float32)


def _add_body(a_ref, b_ref, o_ref):
    o_ref[...] = a_ref[...] + b_ref[...]


def _sc_aggregate(h, src4, dst4, adj4, n, d):
    nwin = src4.shape[1]
    rpt = (n // NS) // 8 * 8
    rem = n - NS * rpt
    zr = 16
    mesh = plsc.VectorSubcoreMesh(core_axis_name="c", subcore_axis_name="s")

    @functools.partial(
        pl.kernel,
        out_type=jax.ShapeDtypeStruct((NC, n, d), jnp.float32),
        mesh=mesh,
        scratch_types=[
            pltpu.VMEM_SHARED((n, d), jnp.float32),
            pltpu.VMEM((WIN, GB), jnp.int32),
            pltpu.VMEM((WIN, GB), jnp.int32),
            pltpu.VMEM((WE // 128, 128), jnp.float32),
            pltpu.VMEM((GB, d), jnp.float32),
            pltpu.VMEM((GB, d), jnp.float32),
            pltpu.VMEM((GB, d), jnp.float32),
            pltpu.VMEM((GB, d), jnp.float32),
            pltpu.VMEM((max(zr, rem), d), jnp.float32),
            pltpu.SemaphoreType.DMA,
            pltpu.SemaphoreType.DMA,
            pltpu.SemaphoreType.DMA,
            pltpu.SemaphoreType.DMA,
            pltpu.SemaphoreType.DMA,
            pltpu.SemaphoreType.DMA,
            pltpu.SemaphoreType.DMA,
            pltpu.SemaphoreType.DMA,
        ],
    )
    def k(h_hbm, src_hbm, dst_hbm, adj_hbm, out_hbm,
          acc, srcv, dstv, adjv, rows0, rows1, rows2, rows3, zbuf,
          g0, g1, g2, g3, s0, s1, s2, s3):
        c = lax.axis_index("c")
        s = lax.axis_index("s")
        wid = s * NC + c
        bufs = (rows0, rows1, rows2, rows3)
        gsems = (g0, g1, g2, g3)
        ssems = (s0, s1, s2, s3)

        def zrow(i, _):
            for j in range(d // LANES):
                zbuf[i, pl.ds(j * LANES, LANES)] = jnp.zeros(
                    (LANES,), jnp.float32)
            return 0
        lax.fori_loop(0, max(zr, rem), zrow, 0)
        my_base = pl.multiple_of(s * rpt, 8)

        def zcopy(r, _):
            off = pl.multiple_of(s * rpt + r * zr, 8)
            pltpu.sync_copy(zbuf, acc.at[pl.ds(off, zr)])
            return 0
        lax.fori_loop(0, rpt // zr, zcopy, 0)
        if rem:
            @pl.when(s == NS - 1)
            def _():
                pltpu.sync_copy(zbuf.at[pl.ds(0, rem)],
                                acc.at[pl.ds(NS * rpt, rem)])
        plsc.subcore_barrier()

        dn = lax.GatherDimensionNumbers(
            offset_dims=(), collapsed_slice_dims=(0,), start_index_map=(0,))

        def scale(buf, u):
            def quad(r, _):
                le = u * GB + r * 4
                arow = le // 128
                acol = le % 128 // LANES * LANES
                av = adjv[arow, pl.ds(acol, LANES)]
                lane0 = le % LANES
                for t in range(4):
                    sc = lax.gather(
                        av, jnp.full((LANES, 1), lane0 + t, jnp.int32),
                        dn, (1,),
                        mode=lax.GatherScatterMode.PROMISE_IN_BOUNDS)
                    e = r * 4 + t
                    for j in range(d // LANES):
                        sl = pl.ds(j * LANES, LANES)
                        buf[e, sl] = buf[e, sl] * sc
                return 0
            lax.fori_loop(0, GB // 4, quad, 0)

        def swait(b):
            pltpu.make_async_copy(bufs[b], acc.at[dstv.at[0]],
                                  ssems[b]).wait()

        def window(kb, _):
            pltpu.sync_copy(src_hbm.at[wid, kb], srcv)
            pltpu.sync_copy(dst_hbm.at[wid, kb], dstv)
            pltpu.sync_copy(adj_hbm.at[wid, kb], adjv)

            gcp = [None] * WIN
            scp = [None] * WIN
            for up in range(NBUF - 1):
                @pl.when(kb > 0)
                def _(up=up):
                    swait(up)
                gcp[up] = pltpu.async_copy(
                    h_hbm.at[srcv.at[up]], bufs[up], gsems[up])

            for u in range(WIN):
                b = u % NBUF
                gcp[u].wait()
                scale(bufs[b], u)
                scp[u] = pltpu.async_copy(
                    bufs[b], acc.at[dstv.at[u]], ssems[b], add=True)
                if u + NBUF - 1 < WIN:
                    nb = (u + NBUF - 1) % NBUF
                    if u == 0:
                        @pl.when(kb > 0)
                        def _():
                            swait(nb)
                    else:
                        scp[u - 1].wait()
                    gcp[u + NBUF - 1] = pltpu.async_copy(
                        h_hbm.at[srcv.at[u + NBUF - 1]],
                        bufs[nb], gsems[nb])
            return 0
        lax.fori_loop(0, nwin, window, 0)

        for u in range(WIN - NBUF, WIN):
            swait(u % NBUF)

        plsc.subcore_barrier()
        pltpu.sync_copy(acc.at[pl.ds(my_base, rpt)],
                        out_hbm.at[c, pl.ds(my_base, rpt)])
        if rem:
            @pl.when(s == NS - 1)
            def _():
                pltpu.sync_copy(acc.at[pl.ds(NS * rpt, rem)],
                                out_hbm.at[c, pl.ds(NS * rpt, rem)])

    return k(h, src4, dst4, adj4)


def kernel(x, edge_index, adj_values, w):
    n, d_in = x.shape
    d_out = w.shape[1]
    e = adj_values.shape[0]

    bm = 1000
    nb = n // bm
    h = pl.pallas_call(
        _mm_body,
        grid=(nb,),
        in_specs=[
            pl.BlockSpec((bm, d_in), lambda i: (i, 0)),
            pl.BlockSpec((d_in, d_out), lambda i: (0, 0)),
        ],
        out_specs=pl.BlockSpec((bm, d_out), lambda i: (i, 0)),
        out_shape=jax.ShapeDtypeStruct((n, d_out), jnp.float32),
    )(x, w)

    dst = edge_index[0]
    src = edge_index[1]
    span = NW * WE
    e_pad = (e + span - 1) // span * span
    if e_pad != e:
        pad = e_pad - e
        src = jnp.concatenate([src, jnp.zeros((pad,), jnp.int32)])
        dst = jnp.concatenate([dst, jnp.zeros((pad,), jnp.int32)])
        adj_values = jnp.concatenate(
            [adj_values, jnp.zeros((pad,), jnp.float32)])
    ew = e_pad // NW
    nwin = ew // WE
    src4 = src.reshape(NW, nwin, WIN, GB)
    dst4 = dst.reshape(NW, nwin, WIN, GB)
    adj4 = adj_values.reshape(NW, nwin, WE // 128, 128)

    partial = _sc_aggregate(h, src4, dst4, adj4, n, d_out)

    out = pl.pallas_call(
        _add_body,
        grid=(nb,),
        in_specs=[
            pl.BlockSpec((bm, d_out), lambda i: (i, 0)),
            pl.BlockSpec((bm, d_out), lambda i: (i, 0)),
        ],
        out_specs=pl.BlockSpec((bm, d_out), lambda i: (i, 0)),
        out_shape=jax.ShapeDtypeStruct((n, d_out), jnp.float32),
    )(partial[0], partial[1])
    return out
